# baseline (speedup 1.0000x reference)
import functools

import numpy as np
import jax
import jax.numpy as jnp
from jax import lax
from jax.experimental import pallas as pl
from jax.experimental.pallas import tpu as pltpu
from jax.experimental.pallas import tpu_sc as plsc

_N = 10000
_E = 320000
_D = 128
_AW = 16
_NC = 2
_NS = 16
_NW = _NC * _NS
_EW = _E // _NW
_K = 80
_NCHUNK = _EW // _K
_NP = 10240
_RPT = _NP // _NS


def _expand_att(att, heads, ch):
  a = att.reshape(heads, ch).astype(jnp.float32)
  m = (jnp.eye(heads, dtype=jnp.float32)[:, None, :] * a[:, :, None])
  m = m.reshape(heads * ch, heads)
  return jnp.pad(m, ((0, 0), (0, _AW - heads)))


def _expand_mat(head_map):
  e = np.zeros((_AW, _D), np.float32)
  for j, h in enumerate(head_map):
    e[h, 16 * j:16 * (j + 1)] = 1.0
  return jnp.asarray(e)


@functools.partial(
    pl.pallas_call,
    out_shape=[
        jax.ShapeDtypeStruct((_N, _D), jnp.float32),
        jax.ShapeDtypeStruct((_N, _AW), jnp.float32),
        jax.ShapeDtypeStruct((_N, _AW), jnp.float32),
    ])
def _dense_in(x_ref, w_ref, as_ref, ad_ref, h_ref, sv_ref, dv_ref):
  h = jnp.dot(x_ref[...], w_ref[...], preferred_element_type=jnp.float32)
  h_ref[...] = h
  sv_ref[...] = jnp.dot(h, as_ref[...], preferred_element_type=jnp.float32)
  dv_ref[...] = jnp.dot(h, ad_ref[...], preferred_element_type=jnp.float32)


@functools.partial(
    pl.pallas_call,
    out_shape=[
        jax.ShapeDtypeStruct((_N, _D), jnp.float32),
        jax.ShapeDtypeStruct((_N, _AW), jnp.float32),
        jax.ShapeDtypeStruct((_N, _AW), jnp.float32),
    ])
def _dense_mid(p_ref, d_ref, e_ref, b_ref, w2_ref, as2_ref, ad2_ref,
               h2_ref, sv_ref, dv_ref):
  acc = p_ref[0, :_N] + p_ref[1, :_N]
  den = jnp.dot(d_ref[0, :_N] + d_ref[1, :_N], e_ref[...],
                preferred_element_type=jnp.float32)
  z = acc / (den + 1e-16) + b_ref[...]
  z = jnp.where(z > 0.0, z, jnp.exp(z) - 1.0)
  h2 = jnp.dot(z, w2_ref[...], preferred_element_type=jnp.float32)
  h2_ref[...] = h2
  sv_ref[...] = jnp.dot(h2, as2_ref[...], preferred_element_type=jnp.float32)
  dv_ref[...] = jnp.dot(h2, ad2_ref[...], preferred_element_type=jnp.float32)


@functools.partial(
    pl.pallas_call,
    out_shape=jax.ShapeDtypeStruct((_N, _D), jnp.float32))
def _dense_out(p_ref, d_ref, e_ref, b_ref, o_ref):
  acc = p_ref[0, :_N] + p_ref[1, :_N]
  den = jnp.dot(d_ref[0, :_N] + d_ref[1, :_N], e_ref[...],
                preferred_element_type=jnp.float32)
  z = acc / (den + 1e-16) + b_ref[...]
  o_ref[...] = jnp.where(z > 0.0, z, jnp.exp(z) - 1.0)


def _make_edge_pass(head_map):
  hm = tuple(head_map)
  mesh = plsc.VectorSubcoreMesh(
      core_axis_name="c", subcore_axis_name="s",
      num_cores=_NC, num_subcores=_NS)

  @functools.partial(
      pl.kernel,
      out_type=[
          jax.ShapeDtypeStruct((_NC, _NP, _D), jnp.float32),
          jax.ShapeDtypeStruct((_NC, _NP, _AW), jnp.float32),
      ],
      mesh=mesh,
      compiler_params=pltpu.CompilerParams(use_tc_tiling_on_sc=False),
      scratch_types=[
          pltpu.VMEM((_K,), jnp.int32),
          pltpu.VMEM((_K,), jnp.int32),
          pltpu.VMEM((_K, _D), jnp.float32),
          pltpu.VMEM((_K, _AW), jnp.float32),
          pltpu.VMEM((_K, _AW), jnp.float32),
          pltpu.VMEM((_K,), jnp.int32),
          pltpu.VMEM((_K,), jnp.int32),
          pltpu.VMEM((_K, _D), jnp.float32),
          pltpu.VMEM((_K, _AW), jnp.float32),
          pltpu.VMEM((_K, _AW), jnp.float32),
          pltpu.VMEM_SHARED((_NP, _D), jnp.float32),
          pltpu.VMEM_SHARED((_NP, _AW), jnp.float32),
          pltpu.SemaphoreType.DMA,
          pltpu.SemaphoreType.DMA,
          pltpu.SemaphoreType.DMA,
          pltpu.SemaphoreType.DMA,
      ])
  def edge_pass(h_hbm, as_hbm, ad_hbm, src_hbm, dst_hbm, zr_hbm, zd_hbm,
                accp_hbm, denp_hbm,
                srcv0, dstv0, rows0, ag0, wg0,
                srcv1, dstv1, rows1, ag1, wg1,
                acc_sh, den_sh, semg0, semg1, sems0, sems1):
    cid = lax.axis_index("c")
    sid = lax.axis_index("s")
    wid = sid * _NC + cid

    rsl = pl.ds(sid * _RPT, _RPT)
    pltpu.sync_copy(zr_hbm, acc_sh.at[rsl])
    pltpu.sync_copy(zd_hbm, den_sh.at[rsl])
    plsc.subcore_barrier()

    ebase = wid * _EW
    buf0 = (srcv0, dstv0, rows0, ag0, wg0, semg0, sems0)
    buf1 = (srcv1, dstv1, rows1, ag1, wg1, semg1, sems1)

    def issue_gathers(ci, b):
      srcv, dstv, rows, ag, wg, semg, _ = b
      base = ebase + ci * _K
      pltpu.sync_copy(src_hbm.at[pl.ds(base, _K)], srcv)
      pltpu.sync_copy(dst_hbm.at[pl.ds(base, _K)], dstv)
      pltpu.async_copy(h_hbm.at[srcv], rows, semg)
      pltpu.async_copy(as_hbm.at[srcv], ag, semg)
      pltpu.async_copy(ad_hbm.at[dstv], wg, semg)

    def wait_gathers(b):
      srcv, dstv, rows, ag, wg, semg, _ = b
      pltpu.make_async_copy(h_hbm.at[srcv], rows, semg).wait()
      pltpu.make_async_copy(as_hbm.at[srcv], ag, semg).wait()
      pltpu.make_async_copy(ad_hbm.at[dstv], wg, semg).wait()

    def compute(b):
      srcv, dstv, rows, ag, wg, _, _ = b
      def body(k, c):
        v = ag[k, :] + wg[k, :]
        v = jnp.where(v > 0.0, v, v * jnp.float32(0.2))
        v = jnp.exp(v)
        wg[k, :] = v
        for j in range(_D // 16):
          sl = pl.ds(16 * j, 16)
          rows[k, sl] = rows[k, sl] * v[hm[j]]
        return c
      lax.fori_loop(0, _K, body, 0, unroll=4)

    def issue_scatters(b):
      srcv, dstv, rows, ag, wg, _, sems = b
      pltpu.async_copy(rows, acc_sh.at[dstv], sems, add=True)
      pltpu.async_copy(wg, den_sh.at[dstv], sems, add=True)

    def wait_scatters(b):
      srcv, dstv, rows, ag, wg, _, sems = b
      pltpu.make_async_copy(rows, acc_sh.at[dstv], sems).wait()
      pltpu.make_async_copy(wg, den_sh.at[dstv], sems).wait()

    issue_gathers(0, buf0)

    def pair(t, carry):
      @pl.when(t > 0)
      def _():
        wait_scatters(buf1)
      issue_gathers(2 * t + 1, buf1)
      wait_gathers(buf0)
      compute(buf0)
      issue_scatters(buf0)
      wait_gathers(buf1)
      compute(buf1)
      issue_scatters(buf1)
      wait_scatters(buf0)
      issue_gathers(2 * t + 2, buf0)
      return carry

    lax.fori_loop(0, _NCHUNK // 2, pair, 0)

    wait_scatters(buf1)
    wait_gathers(buf0)
    compute(buf0)
    issue_scatters(buf0)
    wait_scatters(buf0)
    plsc.subcore_barrier()

    pltpu.sync_copy(acc_sh.at[rsl], accp_hbm.at[cid, rsl])
    pltpu.sync_copy(den_sh.at[rsl], denp_hbm.at[cid, rsl])

  return edge_pass


_edge_l1 = _make_edge_pass(tuple(range(8)))
_edge_l2 = _make_edge_pass((0,) * 8)
_E1 = tuple(range(8))
_E2 = (0,) * 8


def kernel(x, edge_index, W1, att_src1, att_dst1, b1,
           W2, att_src2, att_dst2, b2):
  src = edge_index[0]
  dst = edge_index[1]
  as1 = _expand_att(att_src1, 8, 16)
  ad1 = _expand_att(att_dst1, 8, 16)
  as2 = _expand_att(att_src2, 1, 128)
  ad2 = _expand_att(att_dst2, 1, 128)
  e1 = _expand_mat(_E1)
  e2 = _expand_mat(_E2)
  zr = jnp.zeros((_RPT, _D), jnp.float32)
  zd = jnp.zeros((_RPT, _AW), jnp.float32)

  h1, s1, d1 = _dense_in(x, W1, as1, ad1)
  p1, q1 = _edge_l1(h1, s1, d1, src, dst, zr, zd)
  h2, s2, d2 = _dense_mid(p1, q1, e1, b1.reshape(1, _D), W2, as2, ad2)
  p2, q2 = _edge_l2(h2, s2, d2, src, dst, zr, zd)
  return _dense_out(p2, q2, e2, b2.reshape(1, _D))


# speedup vs baseline: 1.0397x; 1.0397x over previous
import functools

import numpy as np
import jax
import jax.numpy as jnp
from jax import lax
from jax.experimental import pallas as pl
from jax.experimental.pallas import tpu as pltpu
from jax.experimental.pallas import tpu_sc as plsc

_N = 10000
_E = 320000
_D = 128
_AW = 16
_NC = 2
_NS = 16
_NW = _NC * _NS
_EW = _E // _NW
_K = 80
_NCHUNK = _EW // _K
_NP = 10240
_RPT = _NP // _NS


def _expand_att(att, heads, ch):
  a = att.reshape(heads, ch).astype(jnp.float32)
  m = (jnp.eye(heads, dtype=jnp.float32)[:, None, :] * a[:, :, None])
  m = m.reshape(heads * ch, heads)
  return jnp.pad(m, ((0, 0), (0, _AW - heads)))


def _expand_mat(head_map):
  e = np.zeros((_AW, _D), np.float32)
  for j, h in enumerate(head_map):
    e[h, 16 * j:16 * (j + 1)] = 1.0
  return jnp.asarray(e)


@functools.partial(
    pl.pallas_call,
    out_shape=[
        jax.ShapeDtypeStruct((_N, _D), jnp.float32),
        jax.ShapeDtypeStruct((_N, _AW), jnp.float32),
        jax.ShapeDtypeStruct((_N, _AW), jnp.float32),
    ])
def _dense_in(x_ref, w_ref, as_ref, ad_ref, h_ref, sv_ref, dv_ref):
  h = jnp.dot(x_ref[...], w_ref[...], preferred_element_type=jnp.float32)
  h_ref[...] = h
  sv_ref[...] = jnp.dot(h, as_ref[...], preferred_element_type=jnp.float32)
  dv_ref[...] = jnp.dot(h, ad_ref[...], preferred_element_type=jnp.float32)


@functools.partial(
    pl.pallas_call,
    out_shape=[
        jax.ShapeDtypeStruct((_N, _D), jnp.float32),
        jax.ShapeDtypeStruct((_N, _AW), jnp.float32),
        jax.ShapeDtypeStruct((_N, _AW), jnp.float32),
    ])
def _dense_mid(p_ref, d_ref, e_ref, b_ref, w2_ref, as2_ref, ad2_ref,
               h2_ref, sv_ref, dv_ref):
  acc = p_ref[0, :_N] + p_ref[1, :_N]
  den = jnp.dot(d_ref[0, :_N] + d_ref[1, :_N], e_ref[...],
                preferred_element_type=jnp.float32)
  z = acc / (den + 1e-16) + b_ref[...]
  z = jnp.where(z > 0.0, z, jnp.exp(z) - 1.0)
  h2 = jnp.dot(z, w2_ref[...], preferred_element_type=jnp.float32)
  h2_ref[...] = h2
  sv_ref[...] = jnp.dot(h2, as2_ref[...], preferred_element_type=jnp.float32)
  dv_ref[...] = jnp.dot(h2, ad2_ref[...], preferred_element_type=jnp.float32)


@functools.partial(
    pl.pallas_call,
    out_shape=jax.ShapeDtypeStruct((_N, _D), jnp.float32))
def _dense_out(p_ref, d_ref, e_ref, b_ref, o_ref):
  acc = p_ref[0, :_N] + p_ref[1, :_N]
  den = jnp.dot(d_ref[0, :_N] + d_ref[1, :_N], e_ref[...],
                preferred_element_type=jnp.float32)
  z = acc / (den + 1e-16) + b_ref[...]
  o_ref[...] = jnp.where(z > 0.0, z, jnp.exp(z) - 1.0)


def _make_edge_pass(head_map):
  hm = tuple(head_map)
  mesh = plsc.VectorSubcoreMesh(
      core_axis_name="c", subcore_axis_name="s",
      num_cores=_NC, num_subcores=_NS)

  @functools.partial(
      pl.kernel,
      out_type=[
          jax.ShapeDtypeStruct((_NC, _NP, _D), jnp.float32),
          jax.ShapeDtypeStruct((_NC, _NP, _AW), jnp.float32),
      ],
      mesh=mesh,
      compiler_params=pltpu.CompilerParams(use_tc_tiling_on_sc=False),
      scratch_types=[
          pltpu.VMEM((_K,), jnp.int32),
          pltpu.VMEM((_K,), jnp.int32),
          pltpu.VMEM((_K, _D), jnp.float32),
          pltpu.VMEM((_K, _AW), jnp.float32),
          pltpu.VMEM((_K, _AW), jnp.float32),
          pltpu.VMEM((_K,), jnp.int32),
          pltpu.VMEM((_K,), jnp.int32),
          pltpu.VMEM((_K, _D), jnp.float32),
          pltpu.VMEM((_K, _AW), jnp.float32),
          pltpu.VMEM((_K, _AW), jnp.float32),
          pltpu.VMEM_SHARED((_NP, _D), jnp.float32),
          pltpu.VMEM_SHARED((_NP, _AW), jnp.float32),
          pltpu.SemaphoreType.DMA,
          pltpu.SemaphoreType.DMA,
          pltpu.SemaphoreType.DMA,
          pltpu.SemaphoreType.DMA,
      ])
  def edge_pass(h_hbm, as_hbm, ad_hbm, src_hbm, dst_hbm, zr_hbm, zd_hbm,
                accp_hbm, denp_hbm,
                srcv0, dstv0, rows0, ag0, wg0,
                srcv1, dstv1, rows1, ag1, wg1,
                acc_sh, den_sh, semg0, semg1, sems0, sems1):
    cid = lax.axis_index("c")
    sid = lax.axis_index("s")
    wid = sid * _NC + cid

    rsl = pl.ds(sid * _RPT, _RPT)
    pltpu.sync_copy(zr_hbm, acc_sh.at[rsl])
    pltpu.sync_copy(zd_hbm, den_sh.at[rsl])
    plsc.subcore_barrier()

    ebase = wid * _EW
    buf0 = (srcv0, dstv0, rows0, ag0, wg0, semg0, sems0)
    buf1 = (srcv1, dstv1, rows1, ag1, wg1, semg1, sems1)

    def issue_gathers(ci, b):
      srcv, dstv, rows, ag, wg, semg, _ = b
      base = ebase + ci * _K
      pltpu.sync_copy(src_hbm.at[pl.ds(base, _K)], srcv)
      pltpu.sync_copy(dst_hbm.at[pl.ds(base, _K)], dstv)
      pltpu.async_copy(h_hbm.at[srcv], rows, semg)
      pltpu.async_copy(as_hbm.at[srcv], ag, semg)
      pltpu.async_copy(ad_hbm.at[dstv], wg, semg)

    def wait_gathers(b):
      srcv, dstv, rows, ag, wg, semg, _ = b
      pltpu.make_async_copy(h_hbm.at[srcv], rows, semg).wait()
      pltpu.make_async_copy(as_hbm.at[srcv], ag, semg).wait()
      pltpu.make_async_copy(ad_hbm.at[dstv], wg, semg).wait()

    def compute(b):
      srcv, dstv, rows, ag, wg, _, _ = b
      def body(k, c):
        v = ag[k, :] + wg[k, :]
        v = jnp.where(v > 0.0, v, v * jnp.float32(0.2))
        v = jnp.exp(v)
        wg[k, :] = v
        for j in range(_D // 16):
          sl = pl.ds(16 * j, 16)
          rows[k, sl] = rows[k, sl] * v[hm[j]]
        return c
      lax.fori_loop(0, _K, body, 0)

    def issue_scatters(b):
      srcv, dstv, rows, ag, wg, _, sems = b
      pltpu.async_copy(rows, acc_sh.at[dstv], sems, add=True)
      pltpu.async_copy(wg, den_sh.at[dstv], sems, add=True)

    def wait_scatters(b):
      srcv, dstv, rows, ag, wg, _, sems = b
      pltpu.make_async_copy(rows, acc_sh.at[dstv], sems).wait()
      pltpu.make_async_copy(wg, den_sh.at[dstv], sems).wait()

    issue_gathers(0, buf0)

    def pair(t, carry):
      @pl.when(t > 0)
      def _():
        wait_scatters(buf1)
      issue_gathers(2 * t + 1, buf1)
      wait_gathers(buf0)
      compute(buf0)
      issue_scatters(buf0)
      wait_gathers(buf1)
      compute(buf1)
      issue_scatters(buf1)
      wait_scatters(buf0)
      issue_gathers(2 * t + 2, buf0)
      return carry

    lax.fori_loop(0, _NCHUNK // 2, pair, 0)

    wait_scatters(buf1)
    wait_gathers(buf0)
    compute(buf0)
    issue_scatters(buf0)
    wait_scatters(buf0)
    plsc.subcore_barrier()

    pltpu.sync_copy(acc_sh.at[rsl], accp_hbm.at[cid, rsl])
    pltpu.sync_copy(den_sh.at[rsl], denp_hbm.at[cid, rsl])

  return edge_pass


_edge_l1 = _make_edge_pass(tuple(range(8)))
_edge_l2 = _make_edge_pass((0,) * 8)
_E1 = tuple(range(8))
_E2 = (0,) * 8


def kernel(x, edge_index, W1, att_src1, att_dst1, b1,
           W2, att_src2, att_dst2, b2):
  src = edge_index[0]
  dst = edge_index[1]
  as1 = _expand_att(att_src1, 8, 16)
  ad1 = _expand_att(att_dst1, 8, 16)
  as2 = _expand_att(att_src2, 1, 128)
  ad2 = _expand_att(att_dst2, 1, 128)
  e1 = _expand_mat(_E1)
  e2 = _expand_mat(_E2)
  zr = jnp.zeros((_RPT, _D), jnp.float32)
  zd = jnp.zeros((_RPT, _AW), jnp.float32)

  h1, s1, d1 = _dense_in(x, W1, as1, ad1)
  p1, q1 = _edge_l1(h1, s1, d1, src, dst, zr, zd)
  h2, s2, d2 = _dense_mid(p1, q1, e1, b1.reshape(1, _D), W2, as2, ad2)
  p2, q2 = _edge_l2(h2, s2, d2, src, dst, zr, zd)
  return _dense_out(p2, q2, e2, b2.reshape(1, _D))
